# Initial kernel scaffold; baseline (speedup 1.0000x reference)
import jax
import jax.numpy as jnp
from jax.experimental import pallas as pl


def kernel(x, edge_index, W1, b1, W2, b2):
    raise NotImplementedError("write your pallas kernel here")


# speedup vs baseline: 341.2191x; 341.2191x over previous
import functools

import jax
import jax.numpy as jnp
from jax import lax
from jax.experimental import pallas as pl
from jax.experimental.pallas import tpu as pltpu
from jax.experimental.pallas import tpu_sc as plsc

_NC = 2
_NS = 16
_NW = _NC * _NS
_CHUNK = 128
_U = 8
_GRP = _CHUNK * _U


def _ceil_to(a: int, m: int) -> int:
    return -(-a // m) * m


def _i32(v):
    return jnp.int32(v)


def _deg_body(n_outer, dst_hbm, out_hbm, didx, ones_v, zbuf, acc, sem):
    c = lax.axis_index("c")
    s = lax.axis_index("s")
    wid = c * _i32(_NS) + s
    rpt = zbuf.shape[0]

    for j in range(_CHUNK // 16):
        ones_v[pl.ds(j * 16, 16)] = jnp.ones((16,), jnp.float32)

    def zfill(i, carry):
        zbuf[pl.ds(i * _i32(16), 16)] = jnp.zeros((16,), jnp.float32)
        return carry

    lax.fori_loop(_i32(0), _i32(rpt // 16), zfill, _i32(0))

    pltpu.sync_copy(zbuf, acc.at[pl.ds(s * _i32(rpt), rpt)])
    plsc.subcore_barrier()

    ebase = wid * _i32(n_outer * _GRP)

    def body(g, carry):
        base = ebase + g * _i32(_GRP)
        descs = [
            pltpu.async_copy(dst_hbm.at[pl.ds(base + _i32(j * _CHUNK), _CHUNK)],
                             didx[j], sem)
            for j in range(_U)
        ]
        for d in descs:
            d.wait()
        for j in range(_U):
            pltpu.sync_copy(ones_v, acc.at[didx[j]], add=True)
        return carry

    lax.fori_loop(_i32(0), _i32(n_outer), body, _i32(0))
    plsc.subcore_barrier()

    nrows = acc.shape[0]
    pltpu.sync_copy(acc.at[pl.ds(s * _i32(rpt), rpt)], zbuf)
    pltpu.sync_copy(zbuf,
                    out_hbm.at[pl.ds(c * _i32(nrows) + s * _i32(rpt), rpt)])


def _agg_body(n_outer, half, src_hbm, dst_hbm, table_hbm, out_hbm,
              sidx, didx, msgs, zbuf, acc, sem_i, sem_g):
    c = lax.axis_index("c")
    s = lax.axis_index("s")
    rpt = acc.shape[0] // _NS
    hid = zbuf.shape[1]

    def zfill(i, carry):
        zbuf[i, :] = jnp.zeros((hid,), jnp.float32)
        return carry

    lax.fori_loop(_i32(0), _i32(rpt), zfill, _i32(0))
    pltpu.sync_copy(zbuf, acc.at[pl.ds(s * _i32(rpt), rpt), :])
    plsc.subcore_barrier()

    ebase = s * _i32(n_outer * _GRP)
    cbase = c * _i32(half)
    dummy = _i32(half) + lax.iota(jnp.int32, 16)

    def body(g, carry):
        base = ebase + g * _i32(_GRP)
        descs = []
        for j in range(_U):
            descs.append(pltpu.async_copy(
                src_hbm.at[pl.ds(base + _i32(j * _CHUNK), _CHUNK)], sidx[j], sem_i))
            descs.append(pltpu.async_copy(
                dst_hbm.at[pl.ds(base + _i32(j * _CHUNK), _CHUNK)], didx[j], sem_i))
        for d in descs:
            d.wait()
        gd = [pltpu.async_copy(table_hbm.at[sidx[j]], msgs[j], sem_g)
              for j in range(_U)]
        for j in range(_U):
            for v in range(_CHUNK // 16):
                d = didx[j][pl.ds(v * 16, 16)]
                local = d - cbase
                ok = (local >= 0) & (local < _i32(half))
                didx[j][pl.ds(v * 16, 16)] = jnp.where(ok, local, dummy)
        for d in gd:
            d.wait()
        for j in range(_U):
            pltpu.sync_copy(msgs[j], acc.at[didx[j]], add=True)
        return carry

    lax.fori_loop(_i32(0), _i32(n_outer), body, _i32(0))
    plsc.subcore_barrier()

    pltpu.sync_copy(acc.at[pl.ds(s * _i32(rpt), rpt), :], zbuf)
    pltpu.sync_copy(zbuf, out_hbm.at[c, pl.ds(s * _i32(rpt), rpt), :])


def _k1_body(x_ref, w1_ref, dega_ref, degb_ref, h1s_ref, dinv_ref):
    d = dega_ref[...] + degb_ref[...] + 1.0
    dv = lax.rsqrt(d)
    h = jnp.dot(x_ref[...], w1_ref[...], preferred_element_type=jnp.float32)
    h1s_ref[...] = h * dv
    dinv_ref[...] = dv


def _k2_body(p1_ref, h1s_ref, dinv_ref, b1_ref, z1s_ref):
    dv = dinv_ref[...]
    agg = dv * (p1_ref[0] + h1s_ref[...])
    z = jnp.maximum(agg + b1_ref[...], 0.0)
    z1s_ref[...] = z * dv


def _k3_body(p2_ref, z1s_ref, dinv_ref, w2_ref, b2_ref, out_ref):
    dv = dinv_ref[...]
    agg = dv * (p2_ref[0] + z1s_ref[...])
    o = jnp.dot(agg, w2_ref[...], preferred_element_type=jnp.float32)
    o = o + b2_ref[...]
    m = jnp.max(o, axis=1, keepdims=True)
    e = jnp.exp(o - m)
    ssum = jnp.sum(e, axis=1, keepdims=True)
    out_ref[...] = o - m - jnp.log(ssum)


def kernel(x, edge_index, W1, b1, W2, b2):
    N, F = x.shape
    E = edge_index.shape[1]
    HID = W1.shape[1]
    C = W2.shape[1]

    out_dtype = jnp.result_type(x.dtype, W1.dtype, W2.dtype)
    x = x.astype(jnp.float32)
    W1 = W1.astype(jnp.float32)
    W2 = W2.astype(jnp.float32)
    b1 = b1.astype(jnp.float32)
    b2 = b2.astype(jnp.float32)

    ept_deg = _ceil_to(-(-E // _NW), _GRP)
    n_outer_deg = ept_deg // _GRP
    ept_agg = _ceil_to(-(-E // _NS), _GRP)
    n_outer_agg = ept_agg // _GRP
    e_pad = max(_NW * ept_deg, _NS * ept_agg)
    rpt_deg = _ceil_to(-(-(N + 1) // _NS), 8)
    deg_rows = _NS * rpt_deg
    half = N // 2
    rpt_agg = _ceil_to(-(-(half + 16) // _NS), 8)
    agg_rows = _NS * rpt_agg

    src = edge_index[0].astype(jnp.int32)
    dst = edge_index[1].astype(jnp.int32)
    pad = e_pad - E
    src_p = jnp.concatenate([src, jnp.zeros((pad,), jnp.int32)])
    dst_p = jnp.concatenate([dst, jnp.full((pad,), N, jnp.int32)])

    mesh = plsc.VectorSubcoreMesh(core_axis_name="c", subcore_axis_name="s")

    deg_call = pl.kernel(
        functools.partial(_deg_body, n_outer_deg),
        out_type=jax.ShapeDtypeStruct((_NC * deg_rows,), jnp.float32),
        mesh=mesh,
        compiler_params=pltpu.CompilerParams(use_tc_tiling_on_sc=False),
        scratch_types=[
            [pltpu.VMEM((_CHUNK,), jnp.int32) for _ in range(_U)],
            pltpu.VMEM((_CHUNK,), jnp.float32),
            pltpu.VMEM((rpt_deg,), jnp.float32),
            pltpu.VMEM_SHARED((deg_rows,), jnp.float32),
            pltpu.SemaphoreType.DMA,
        ],
    )
    deg2 = deg_call(dst_p)

    agg_call = pl.kernel(
        functools.partial(_agg_body, n_outer_agg, half),
        out_type=jax.ShapeDtypeStruct((_NC, agg_rows, HID), jnp.float32),
        mesh=mesh,
        compiler_params=pltpu.CompilerParams(use_tc_tiling_on_sc=False),
        scratch_types=[
            [pltpu.VMEM((_CHUNK,), jnp.int32) for _ in range(_U)],
            [pltpu.VMEM((_CHUNK,), jnp.int32) for _ in range(_U)],
            [pltpu.VMEM((_CHUNK, HID), jnp.float32) for _ in range(_U)],
            pltpu.VMEM((rpt_agg, HID), jnp.float32),
            pltpu.VMEM_SHARED((agg_rows, HID), jnp.float32),
            pltpu.SemaphoreType.DMA,
            pltpu.SemaphoreType.DMA,
        ],
    )

    RB = 2000
    NB = N // RB
    BPC = half // RB
    dega = jnp.reshape(deg2[:deg_rows], (deg_rows, 1))
    degb = jnp.reshape(deg2[deg_rows:], (deg_rows, 1))

    h1s, dinv = pl.pallas_call(
        _k1_body,
        grid=(NB,),
        in_specs=[
            pl.BlockSpec((RB, F), lambda i: (i, _i32(0))),
            pl.BlockSpec((F, HID), lambda i: (_i32(0), _i32(0))),
            pl.BlockSpec((RB, 1), lambda i: (i, _i32(0))),
            pl.BlockSpec((RB, 1), lambda i: (i, _i32(0))),
        ],
        out_specs=[
            pl.BlockSpec((RB, HID), lambda i: (i, _i32(0))),
            pl.BlockSpec((RB, 1), lambda i: (i, _i32(0))),
        ],
        out_shape=[
            jax.ShapeDtypeStruct((N, HID), jnp.float32),
            jax.ShapeDtypeStruct((N, 1), jnp.float32),
        ],
    )(x, W1, dega, degb)

    p1 = agg_call(src_p, dst_p, h1s)

    p_spec = pl.BlockSpec(
        (1, RB, HID),
        lambda i: (lax.div(i, _i32(BPC)), lax.rem(i, _i32(BPC)), _i32(0)))

    b1r = jnp.reshape(b1, (1, HID))
    z1s = pl.pallas_call(
        _k2_body,
        grid=(NB,),
        in_specs=[
            p_spec,
            pl.BlockSpec((RB, HID), lambda i: (i, _i32(0))),
            pl.BlockSpec((RB, 1), lambda i: (i, _i32(0))),
            pl.BlockSpec((1, HID), lambda i: (_i32(0), _i32(0))),
        ],
        out_specs=pl.BlockSpec((RB, HID), lambda i: (i, _i32(0))),
        out_shape=jax.ShapeDtypeStruct((N, HID), jnp.float32),
    )(p1, h1s, dinv, b1r)

    p2 = agg_call(src_p, dst_p, z1s)

    b2r = jnp.reshape(b2, (1, C))
    out = pl.pallas_call(
        _k3_body,
        grid=(NB,),
        in_specs=[
            p_spec,
            pl.BlockSpec((RB, HID), lambda i: (i, _i32(0))),
            pl.BlockSpec((RB, 1), lambda i: (i, _i32(0))),
            pl.BlockSpec((HID, C), lambda i: (_i32(0), _i32(0))),
            pl.BlockSpec((1, C), lambda i: (_i32(0), _i32(0))),
        ],
        out_specs=pl.BlockSpec((RB, C), lambda i: (i, _i32(0))),
        out_shape=jax.ShapeDtypeStruct((N, C), jnp.float32),
    )(p2, z1s, dinv, W2, b2r)

    return out.astype(out_dtype)
